# baseline (speedup 1.0000x reference)
import functools

import jax
import jax.numpy as jnp
from jax import lax
from jax.experimental import pallas as pl
from jax.experimental.pallas import tpu as pltpu
from jax.experimental.pallas import tpu_sc as plsc

_BLOCK_ROWS = 256
_NC, _NS, _L = 2, 16, 16
_NW = _NC * _NS


def _copy_body(x_ref, o_ref):
    o_ref[...] = x_ref[...]


@functools.lru_cache(maxsize=None)
def _make_sc_patch(n):
    per_w = n // _NW
    chunk = 128 if per_w % 128 == 0 else per_w
    n_chunks = per_w // chunk
    n_vregs = chunk // _L

    @functools.partial(
        pl.kernel,
        out_type=(),
        mesh=plsc.VectorSubcoreMesh(core_axis_name="c", subcore_axis_name="s"),
        scratch_types=[
            pltpu.VMEM((chunk,), jnp.int32),
            pltpu.VMEM((chunk,), jnp.float32),
            pltpu.VMEM((_L,), jnp.float32),
            pltpu.SemaphoreType.DMA,
        ],
    )
    def sc_patch(y_hbm, sig_hbm, idx_v, vals_v, sig_v, sem):
        wid = lax.axis_index("s") * _NC + lax.axis_index("c")
        pltpu.sync_copy(sig_hbm, sig_v)
        sig = sig_v[...]
        stride = jnp.int32(n + 1)
        for c in range(n_chunks):
            base = wid * per_w + c * chunk
            for t in range(n_vregs):
                gi = base + t * _L + lax.iota(jnp.int32, _L)
                idx_v[pl.ds(t * _L, _L)] = gi * stride
            pltpu.async_copy(y_hbm.at[idx_v], vals_v, sem).wait()
            for t in range(n_vregs):
                vals_v[pl.ds(t * _L, _L)] = vals_v[pl.ds(t * _L, _L)] + sig
            pltpu.async_copy(vals_v, y_hbm.at[idx_v], sem).wait()

    return sc_patch


def kernel(x, sigma2):
    n, m = x.shape
    br = _BLOCK_ROWS if n % _BLOCK_ROWS == 0 else n
    y = pl.pallas_call(
        _copy_body,
        grid=(n // br,),
        in_specs=[pl.BlockSpec((br, m), lambda i: (i, 0))],
        out_specs=pl.BlockSpec((br, m), lambda i: (i, 0)),
        out_shape=jax.ShapeDtypeStruct((n, m), x.dtype),
        compiler_params=pltpu.CompilerParams(
            dimension_semantics=("parallel",),
        ),
    )(x)
    sig16 = jnp.broadcast_to(sigma2.astype(x.dtype), (_L,))
    y_ref = jax.new_ref(y.reshape(n * m))
    _make_sc_patch(n)(y_ref, sig16)
    return y_ref[...].reshape(n, m)


# speedup vs baseline: 1.7194x; 1.7194x over previous
import functools

import jax
import jax.numpy as jnp
from jax import lax
from jax.experimental import pallas as pl
from jax.experimental.pallas import tpu as pltpu
from jax.experimental.pallas import tpu_sc as plsc

_BLOCK_ROWS = 256
_NC, _NS, _L = 2, 16, 16
_NW = _NC * _NS


@functools.lru_cache(maxsize=None)
def _make_sc_diag(n):
    per_w = n // _NW
    chunk = 128 if per_w % 128 == 0 else per_w
    n_chunks = per_w // chunk
    n_vregs = chunk // _L

    @functools.partial(
        pl.kernel,
        out_type=jax.ShapeDtypeStruct((n,), jnp.float32),
        mesh=plsc.VectorSubcoreMesh(core_axis_name="c", subcore_axis_name="s"),
        scratch_types=[
            pltpu.VMEM((chunk,), jnp.int32),
            pltpu.VMEM((chunk,), jnp.float32),
            pltpu.VMEM((_L,), jnp.float32),
            pltpu.SemaphoreType.DMA,
        ],
    )
    def sc_diag(x_hbm, sig_hbm, d_hbm, idx_v, vals_v, sig_v, sem):
        wid = lax.axis_index("s") * _NC + lax.axis_index("c")
        pltpu.sync_copy(sig_hbm, sig_v)
        sig = sig_v[...]
        stride = jnp.int32(n + 1)
        for c in range(n_chunks):
            base = wid * per_w + c * chunk
            for t in range(n_vregs):
                gi = base + t * _L + lax.iota(jnp.int32, _L)
                idx_v[pl.ds(t * _L, _L)] = gi * stride
            pltpu.async_copy(x_hbm.at[idx_v], vals_v, sem).wait()
            for t in range(n_vregs):
                vals_v[pl.ds(t * _L, _L)] = vals_v[pl.ds(t * _L, _L)] + sig
            pltpu.sync_copy(vals_v, d_hbm.at[pl.ds(base, chunk)])

    return sc_diag


def _merge_body(x_ref, d_ref, o_ref):
    i = pl.program_id(0)
    o_ref[...] = x_ref[...]
    br = o_ref.shape[0]
    off = i * br
    sub = x_ref[:, pl.ds(off, br)]
    r = jax.lax.broadcasted_iota(jnp.int32, (br, br), 0)
    c = jax.lax.broadcasted_iota(jnp.int32, (br, br), 1)
    dcol = jnp.broadcast_to(d_ref[0], (br, br))
    o_ref[:, pl.ds(off, br)] = jnp.where(r == c, dcol, sub)


def kernel(x, sigma2):
    n, m = x.shape
    br = _BLOCK_ROWS if n % _BLOCK_ROWS == 0 else n
    sig16 = jnp.broadcast_to(sigma2.astype(x.dtype), (_L,))
    d = _make_sc_diag(n)(x.reshape(n * m), sig16)
    d3 = d.reshape(n // br, br, 1)
    return pl.pallas_call(
        _merge_body,
        grid=(n // br,),
        in_specs=[
            pl.BlockSpec((br, m), lambda i: (i, 0)),
            pl.BlockSpec((1, br, 1), lambda i: (i, 0, 0)),
        ],
        out_specs=pl.BlockSpec((br, m), lambda i: (i, 0)),
        out_shape=jax.ShapeDtypeStruct((n, m), x.dtype),
        compiler_params=pltpu.CompilerParams(
            dimension_semantics=("arbitrary",),
        ),
    )(x, d3)


# speedup vs baseline: 3.4070x; 1.9815x over previous
import functools

import jax
import jax.numpy as jnp
from jax import lax
from jax.experimental import pallas as pl
from jax.experimental.pallas import tpu as pltpu
from jax.experimental.pallas import tpu_sc as plsc

_BLOCK_ROWS = 256
_NC, _NS, _L = 2, 16, 16
_NW = _NC * _NS
_CHUNK = 128


def _copy_body(x_ref, o_ref):
    o_ref[...] = x_ref[...]


@functools.lru_cache(maxsize=None)
def _make_sc_patch(n):
    per_w = n // _NW
    chunk = _CHUNK if per_w % _CHUNK == 0 else per_w
    n_chunks = per_w // chunk
    n_vregs = chunk // _L

    @functools.partial(
        pl.kernel,
        out_type=(),
        mesh=plsc.VectorSubcoreMesh(core_axis_name="c", subcore_axis_name="s"),
        scratch_types=[
            pltpu.VMEM((chunk,), jnp.int32),
            pltpu.VMEM((chunk, chunk), jnp.float32),
            pltpu.VMEM((_L,), jnp.float32),
            pltpu.SemaphoreType.DMA,
        ],
    )
    def sc_patch(y_hbm, sig_hbm, idx_v, rows_v, sig_v, sem):
        wid = lax.axis_index("s") * _NC + lax.axis_index("c")
        pltpu.sync_copy(sig_hbm, sig_v)
        sig = sig_v[...]
        lane = lax.iota(jnp.int32, _L)
        for c in range(n_chunks):
            base = wid * per_w + c * chunk
            for t in range(n_vregs):
                kv = t * _L + lax.iota(jnp.int32, _L)
                idx_v[pl.ds(t * _L, _L)] = base + kv
            pltpu.async_copy(
                y_hbm.at[idx_v, pl.ds(base, chunk)], rows_v, sem
            ).wait()
            def _bump(k, carry):
                s0 = (k // _L) * _L
                vec = rows_v[k, pl.ds(s0, _L)]
                rows_v[k, pl.ds(s0, _L)] = vec + jnp.where(
                    lane == k - s0, sig, jnp.float32(0.0)
                )
                return carry

            lax.fori_loop(0, chunk, _bump, 0)
            pltpu.async_copy(
                rows_v, y_hbm.at[idx_v, pl.ds(base, chunk)], sem
            ).wait()

    return sc_patch


def kernel(x, sigma2):
    n, m = x.shape
    br = _BLOCK_ROWS if n % _BLOCK_ROWS == 0 else n
    y = pl.pallas_call(
        _copy_body,
        grid=(n // br,),
        in_specs=[pl.BlockSpec((br, m), lambda i: (i, 0))],
        out_specs=pl.BlockSpec((br, m), lambda i: (i, 0)),
        out_shape=jax.ShapeDtypeStruct((n, m), x.dtype),
        compiler_params=pltpu.CompilerParams(
            dimension_semantics=("parallel",),
        ),
    )(x)
    sig16 = jnp.broadcast_to(sigma2.astype(x.dtype), (_L,))
    y_ref = jax.new_ref(y)
    _make_sc_patch(n)(y_ref, sig16)
    return y_ref[...]


# speedup vs baseline: 3.4689x; 1.0182x over previous
import functools

import jax
import jax.numpy as jnp
from jax import lax
from jax.experimental import pallas as pl
from jax.experimental.pallas import tpu as pltpu
from jax.experimental.pallas import tpu_sc as plsc

_BLOCK_ROWS = 256
_NC, _NS, _L = 2, 16, 16
_NW = _NC * _NS
_CHUNK = 128


def _copy_body(x_ref, o_ref):
    o_ref[...] = x_ref[...]


@functools.lru_cache(maxsize=None)
def _make_sc_patch(n):
    per_w = n // _NW
    chunk = _CHUNK if per_w % _CHUNK == 0 else per_w
    n_chunks = per_w // chunk

    @functools.partial(
        pl.kernel,
        out_type=(),
        mesh=plsc.VectorSubcoreMesh(core_axis_name="c", subcore_axis_name="s"),
        scratch_types=[
            pltpu.VMEM((n_chunks, chunk, chunk), jnp.float32),
            pltpu.VMEM((_L,), jnp.float32),
            pltpu.SemaphoreType.DMA,
        ],
    )
    def sc_patch(y_hbm, sig_hbm, blocks_v, sig_v, sem):
        wid = lax.axis_index("s") * _NC + lax.axis_index("c")
        pltpu.sync_copy(sig_hbm, sig_v)
        sig = sig_v[...]
        lane = lax.iota(jnp.int32, _L)
        gathers = []
        for c in range(n_chunks):
            base = wid * per_w + c * chunk
            gathers.append(
                pltpu.async_copy(
                    y_hbm.at[pl.ds(base, chunk), pl.ds(base, chunk)],
                    blocks_v.at[c],
                    sem,
                )
            )
        scatters = []
        for c in range(n_chunks):
            gathers[c].wait()

            def _bump(k, carry):
                s0 = (k // _L) * _L
                vec = blocks_v[c, k, pl.ds(s0, _L)]
                blocks_v[c, k, pl.ds(s0, _L)] = vec + jnp.where(
                    lane == k - s0, sig, jnp.float32(0.0)
                )
                return carry

            lax.fori_loop(0, chunk, _bump, 0)
            base = wid * per_w + c * chunk
            scatters.append(
                pltpu.async_copy(
                    blocks_v.at[c],
                    y_hbm.at[pl.ds(base, chunk), pl.ds(base, chunk)],
                    sem,
                )
            )
        for cp in scatters:
            cp.wait()

    return sc_patch


def kernel(x, sigma2):
    n, m = x.shape
    br = _BLOCK_ROWS if n % _BLOCK_ROWS == 0 else n
    y = pl.pallas_call(
        _copy_body,
        grid=(n // br,),
        in_specs=[pl.BlockSpec((br, m), lambda i: (i, 0))],
        out_specs=pl.BlockSpec((br, m), lambda i: (i, 0)),
        out_shape=jax.ShapeDtypeStruct((n, m), x.dtype),
        compiler_params=pltpu.CompilerParams(
            dimension_semantics=("parallel",),
        ),
    )(x)
    sig16 = jnp.broadcast_to(sigma2.astype(x.dtype), (_L,))
    y_ref = jax.new_ref(y)
    _make_sc_patch(n)(y_ref, sig16)
    return y_ref[...]


# speedup vs baseline: 3.4911x; 1.0064x over previous
import functools

import jax
import jax.numpy as jnp
from jax import lax
from jax.experimental import pallas as pl
from jax.experimental.pallas import tpu as pltpu
from jax.experimental.pallas import tpu_sc as plsc

_BLOCK_ROWS = 256
_NC, _NS, _L = 2, 16, 16
_NW = _NC * _NS
_CHUNK = 128


def _copy_body(x_ref, o_ref):
    o_ref[...] = x_ref[...]


@functools.lru_cache(maxsize=None)
def _make_sc_patch(n):
    per_w = n // _NW
    chunk = _CHUNK if per_w % _CHUNK == 0 else per_w
    n_chunks = per_w // chunk

    @functools.partial(
        pl.kernel,
        out_type=(),
        mesh=plsc.VectorSubcoreMesh(core_axis_name="c", subcore_axis_name="s"),
        scratch_types=[
            pltpu.VMEM((n_chunks, chunk, chunk), jnp.float32),
            pltpu.VMEM((_L,), jnp.float32),
            pltpu.SemaphoreType.DMA,
        ],
    )
    def sc_patch(y_hbm, sig_hbm, blocks_v, sig_v, sem):
        wid = lax.axis_index("s") * _NC + lax.axis_index("c")
        gathers = []
        for c in range(n_chunks):
            base = wid * per_w + c * chunk
            gathers.append(
                pltpu.async_copy(
                    y_hbm.at[pl.ds(base, chunk), pl.ds(base, chunk)],
                    blocks_v.at[c],
                    sem,
                )
            )
        pltpu.sync_copy(sig_hbm, sig_v)
        sig = sig_v[...]
        lane = lax.iota(jnp.int32, _L)
        scatters = []
        for c in range(n_chunks):
            gathers[c].wait()
            for k in range(chunk):
                s0 = (k // _L) * _L
                vec = blocks_v[c, k, pl.ds(s0, _L)]
                blocks_v[c, k, pl.ds(s0, _L)] = vec + jnp.where(
                    lane == k - s0, sig, jnp.float32(0.0)
                )
            base = wid * per_w + c * chunk
            scatters.append(
                pltpu.async_copy(
                    blocks_v.at[c],
                    y_hbm.at[pl.ds(base, chunk), pl.ds(base, chunk)],
                    sem,
                )
            )
        for cp in scatters:
            cp.wait()

    return sc_patch


def kernel(x, sigma2):
    n, m = x.shape
    br = _BLOCK_ROWS if n % _BLOCK_ROWS == 0 else n
    y = pl.pallas_call(
        _copy_body,
        grid=(n // br,),
        in_specs=[pl.BlockSpec((br, m), lambda i: (i, 0))],
        out_specs=pl.BlockSpec((br, m), lambda i: (i, 0)),
        out_shape=jax.ShapeDtypeStruct((n, m), x.dtype),
        compiler_params=pltpu.CompilerParams(
            dimension_semantics=("parallel",),
        ),
    )(x)
    sig16 = jnp.broadcast_to(sigma2.astype(x.dtype), (_L,))
    y_ref = jax.new_ref(y)
    _make_sc_patch(n)(y_ref, sig16)
    return y_ref[...]
